# baseline (speedup 1.0000x reference)
import functools
import math

import numpy as np

import jax
import jax.numpy as jnp
from jax import lax
from jax.experimental import pallas as pl
from jax.experimental.pallas import tpu as pltpu
from jax.experimental.pallas import tpu_sc as plsc

_N = 50000
_E = 800000
_D = 48
_DW = 128
_D0 = 24
_NC = 2
_NS = 16
_NW = _NC * _NS
_NPAD = 50176
_RPT = _NPAD // _NS

_L = 128
_GR = 5
_SGR = 8
_BE = 6400
_E1 = 409600
_E2 = _E - _E1

_INV_SQRT3 = 1.0 / math.sqrt(3.0)


def _gather_body(grpt, gextra):
    gnit = grpt // _GR
    assert gnit * _GR == grpt

    def body_fn(tab, idx2, out, idxb, rowb, sem):
        wid = lax.axis_index("s") * _NC + lax.axis_index("c")
        base_row = wid * grpt
        pltpu.sync_copy(idx2.at[pl.ds(base_row, grpt)],
                        idxb.at[pl.ds(0, grpt)])

        @pl.when(wid < gextra)
        def _():
            pltpu.sync_copy(idx2.at[pl.ds(grpt * _NW + wid, 1)],
                            idxb.at[pl.ds(grpt, 1)])

        def body(g, carry):
            row = g * _GR
            copies = [
                pltpu.async_copy(tab.at[idxb.at[row + k]],
                                 rowb.at[pl.ds(k * _L, _L)], sem)
                for k in range(_GR)
            ]
            for cp in copies:
                cp.wait()
            pltpu.sync_copy(rowb.at[pl.ds(0, _GR * _L), pl.ds(0, _D)],
                            out.at[pl.ds((base_row + row) * _L, _GR * _L),
                                   pl.ds(0, _D)])
            return carry

        lax.fori_loop(0, gnit, body, 0)

        @pl.when(wid < gextra)
        def _():
            pltpu.async_copy(tab.at[idxb.at[grpt]],
                             rowb.at[pl.ds(0, _L)], sem).wait()
            pltpu.sync_copy(rowb.at[pl.ds(0, _L), pl.ds(0, _D)],
                            out.at[pl.ds((grpt * _NW + wid) * _L, _L),
                                   pl.ds(0, _D)])

    return body_fn


def _scatter_body(rows, srpt, snit, snit_last, tail):
    def body_fn(y, dst2, z, o0, o1, idxb, yb, acc, sem):
        c = lax.axis_index("c")
        s = lax.axis_index("s")
        r0 = s * _RPT
        base_row = s * srpt
        nblk = jnp.where(s == _NS - 1, snit_last, snit)

        pltpu.sync_copy(z.at[pl.ds(r0, _RPT)], acc.at[pl.ds(r0, _RPT)])
        plsc.subcore_barrier()

        def _scatter_from(cbase):
            def body(g, carry):
                row = base_row + g * _SGR
                pltpu.sync_copy(dst2.at[pl.ds(row, _SGR)], idxb)
                pltpu.sync_copy(
                    y.at[pl.ds(row * _L, _SGR * _L), pl.ds(cbase, _D0)], yb)
                copies = [
                    pltpu.async_copy(yb.at[pl.ds(k * _L, _L)],
                                     acc.at[idxb.at[k]], sem, add=True)
                    for k in range(_SGR)
                ]
                for cp in copies:
                    cp.wait()
                return carry

            lax.fori_loop(0, nblk, body, 0)

            if tail:
                @pl.when(s == _NS - 1)
                def _():
                    row = rows - tail
                    pltpu.sync_copy(dst2.at[pl.ds(row, tail)],
                                    idxb.at[pl.ds(0, tail)])
                    pltpu.sync_copy(
                        y.at[pl.ds(row * _L, tail * _L), pl.ds(cbase, _D0)],
                        yb.at[pl.ds(0, tail * _L)])
                    copies = [
                        pltpu.async_copy(yb.at[pl.ds(k * _L, _L)],
                                         acc.at[idxb.at[k]], sem, add=True)
                        for k in range(tail)
                    ]
                    for cp in copies:
                        cp.wait()

        @pl.when(c == 0)
        def _():
            _scatter_from(0)

        @pl.when(c == 1)
        def _():
            _scatter_from(_D0)

        plsc.subcore_barrier()

        @pl.when(c == 0)
        def _():
            pltpu.sync_copy(acc.at[pl.ds(r0, _RPT)], o0.at[pl.ds(r0, _RPT)])

        @pl.when(c == 1)
        def _():
            pltpu.sync_copy(acc.at[pl.ds(r0, _RPT)], o1.at[pl.ds(r0, _RPT)])

    return body_fn


@functools.cache
def _make_sc_kernels(e_chunk):
    mesh = plsc.VectorSubcoreMesh(
        core_axis_name="c", subcore_axis_name="s",
        num_cores=_NC, num_subcores=_NS,
    )
    params = pltpu.CompilerParams(use_tc_tiling_on_sc=False)
    rows = e_chunk // _L
    grpt = (rows // _NW) - (rows // _NW) % _GR
    gextra = rows - grpt * _NW
    assert 0 <= gextra <= _NW
    srpt = rows // _NS
    srpt += (-srpt) % _SGR
    snit = srpt // _SGR
    last_rows = rows - (_NS - 1) * srpt
    assert last_rows > 0
    snit_last = last_rows // _SGR
    tail = last_rows - snit_last * _SGR

    gather = functools.partial(
        pl.kernel,
        out_type=jax.ShapeDtypeStruct((e_chunk, _DW), jnp.float32),
        mesh=mesh,
        compiler_params=params,
        scratch_types=[
            pltpu.VMEM((grpt + 1, _L), jnp.int32),
            pltpu.VMEM((_GR * _L, _DW), jnp.float32),
            pltpu.SemaphoreType.DMA,
        ],
    )(_gather_body(grpt, gextra))
    scatter = functools.partial(
        pl.kernel,
        out_type=(
            jax.ShapeDtypeStruct((_NPAD, _D0), jnp.float32),
            jax.ShapeDtypeStruct((_NPAD, _D0), jnp.float32),
        ),
        mesh=mesh,
        compiler_params=params,
        scratch_types=[
            pltpu.VMEM((_SGR, _L), jnp.int32),
            pltpu.VMEM((_SGR * _L, _D0), jnp.float32),
            pltpu.VMEM_SHARED((_NPAD, _D0), jnp.float32),
            pltpu.SemaphoreType.DMA,
        ],
    )(_scatter_body(rows, srpt, snit, snit_last, tail))
    return gather, scatter


def _bf(x):
    return x.astype(jnp.bfloat16)


def _selperm():
    cols = np.concatenate([np.arange(16), 16 + 3 * np.arange(8),
                           17 + 3 * np.arange(8), 18 + 3 * np.arange(8)])
    sel = np.zeros((40, _DW), np.float32)
    sel[np.arange(40), cols] = 1.0
    return jnp.asarray(sel)


def _tc_body(attr, srcb, i24, selp, w1t, w2t, w3t, wst, wvt, y01):
    at = lax.dot_general(_bf(i24[:]), _bf(attr[...]), (((1,), (1,)), ((), ())),
                         preferred_element_type=jnp.float32)
    st = lax.dot_general(_bf(selp[:, 0:40]), _bf(srcb[...][:, 0:40]),
                         (((1,), (1,)), ((), ())),
                         preferred_element_type=jnp.float32)
    inv = at[0:16, :]
    h = jnp.maximum(
        jnp.dot(_bf(w1t[:]), _bf(inv), preferred_element_type=jnp.float32) * 0.25,
        0.0)
    h = jnp.maximum(
        jnp.dot(_bf(w2t[:]), _bf(h), preferred_element_type=jnp.float32) * 0.125,
        0.0)
    f = jnp.dot(_bf(w3t[:]), _bf(h), preferred_element_type=jnp.float32) * 0.125
    fs = f[0:24, :]
    fv = f[24:48, :]
    ex = at[16:17, :]
    ey = at[17:18, :]
    ez = at[18:19, :]
    cw = at[19:20, :]
    s_ = st[0:16, :]
    vx = st[16:24, :]
    vy = st[24:32, :]
    vz = st[32:40, :]
    tp0 = (vx * ex + vy * ey + vz * ez) * _INV_SQRT3
    ms = jnp.concatenate([tp0, s_], axis=0) * fs * cw
    mx = jnp.concatenate([s_ * ex, vx], axis=0) * fv * cw
    my = jnp.concatenate([s_ * ey, vy], axis=0) * fv * cw
    mz = jnp.concatenate([s_ * ez, vz], axis=0) * fv * cw
    ys = jnp.dot(_bf(wst[:]), _bf(ms), preferred_element_type=jnp.float32)
    yx = jnp.dot(_bf(wvt[:]), _bf(mx), preferred_element_type=jnp.float32)
    yy = jnp.dot(_bf(wvt[:]), _bf(my), preferred_element_type=jnp.float32)
    yz = jnp.dot(_bf(wvt[:]), _bf(mz), preferred_element_type=jnp.float32)
    yt = jnp.concatenate([ys, yx, yy, yz], axis=0)
    y01[...] = lax.dot_general(yt, selp[:], (((0,), (0,)), ((), ())),
                               preferred_element_type=jnp.float32)


@functools.cache
def _make_tc(e_chunk):
    return pl.pallas_call(
        _tc_body,
        grid=(e_chunk // _BE,),
        in_specs=[
            pl.BlockSpec((_BE, 24), lambda i: (i, 0)),
            pl.BlockSpec((_BE, _DW), lambda i: (i, 0)),
            pl.BlockSpec((24, 24), lambda i: (0, 0)),
            pl.BlockSpec((40, _DW), lambda i: (0, 0)),
            pl.BlockSpec((64, 16), lambda i: (0, 0)),
            pl.BlockSpec((64, 64), lambda i: (0, 0)),
            pl.BlockSpec((48, 64), lambda i: (0, 0)),
            pl.BlockSpec((16, 24), lambda i: (0, 0)),
            pl.BlockSpec((8, 24), lambda i: (0, 0)),
        ],
        out_specs=pl.BlockSpec((_BE, 128), lambda i: (i, 0)),
        out_shape=jax.ShapeDtypeStruct((e_chunk, 128), jnp.float32),
        compiler_params=pltpu.CompilerParams(
            fuse_transposed_lhs_in_matmul=True),
    )


def kernel(edge_src, edge_dst, edge_weight_cutoff, edge_attr, node_feat,
           W1, W2, W3, Ws, Wv):
    node_tab = jnp.concatenate(
        [node_feat, jnp.zeros((_N, _DW - 40), jnp.float32)], axis=1)
    src_idx = edge_src.astype(jnp.int32).reshape(_E // _L, _L)
    dst_idx = edge_dst.astype(jnp.int32).reshape(_E // _L, _L)
    attr_all = jnp.concatenate(
        [edge_attr, edge_weight_cutoff[:, None],
         jnp.zeros((_E, 4), jnp.float32)], axis=1)
    selp = _selperm()
    i24 = jnp.eye(24, dtype=jnp.float32)
    z = jnp.zeros((_NPAD, _D0), jnp.float32)

    outs = []
    for e0, e_chunk in ((0, _E1), (_E1, _E2)):
        sc_gather, sc_scatter = _make_sc_kernels(e_chunk)
        r0 = e0 // _L
        src_feat = sc_gather(node_tab, src_idx[r0:r0 + e_chunk // _L])
        y = _make_tc(e_chunk)(attr_all[e0:e0 + e_chunk], src_feat,
                              i24, selp, W1.T, W2.T, W3.T, Ws.T, Wv.T)
        outs.append(sc_scatter(y, dst_idx[r0:r0 + e_chunk // _L], z))

    o0 = outs[0][0] + outs[1][0]
    o1 = outs[0][1] + outs[1][1]
    return jnp.concatenate([o0[:_N, :], o1[:_N, :16]], axis=1)


# speedup vs baseline: 1.0759x; 1.0759x over previous
import functools
import math

import numpy as np

import jax
import jax.numpy as jnp
from jax import lax
from jax.experimental import pallas as pl
from jax.experimental.pallas import tpu as pltpu
from jax.experimental.pallas import tpu_sc as plsc

_N = 50000
_E = 800000
_D = 48
_DW = 128
_D0 = 24
_D1 = 16
_NC = 2
_NS = 16
_NW = _NC * _NS
_NPAD = 50176
_RPT = _NPAD // _NS

_L = 128
_EROWS = _E // _L
_GRPT = _EROWS // _NW
_GEXTRA = _EROWS - _GRPT * _NW
_GR = 5
_GNIT = _GRPT // _GR
_SGR = 8
_SRPT = 392
_SNIT = 49
_SNIT_LAST = 46


@functools.cache
def _make_sc_kernels():
    mesh = plsc.VectorSubcoreMesh(
        core_axis_name="c", subcore_axis_name="s",
        num_cores=_NC, num_subcores=_NS,
    )
    params = pltpu.CompilerParams(use_tc_tiling_on_sc=False)
    gather = functools.partial(
        pl.kernel,
        out_type=jax.ShapeDtypeStruct((_E, _DW), jnp.float32),
        mesh=mesh,
        compiler_params=params,
        scratch_types=[
            pltpu.VMEM((_GRPT + 1, _L), jnp.int32),
            pltpu.VMEM((_GR * _L, _DW), jnp.float32),
            pltpu.SemaphoreType.DMA,
        ],
    )(_sc_gather_body)
    scatter = functools.partial(
        pl.kernel,
        out_type=(
            jax.ShapeDtypeStruct((_NPAD, _D0), jnp.float32),
            jax.ShapeDtypeStruct((_NPAD, _D0), jnp.float32),
        ),
        mesh=mesh,
        compiler_params=params,
        scratch_types=[
            pltpu.VMEM((_SGR, _L), jnp.int32),
            pltpu.VMEM((_SGR * _L, _D0), jnp.float32),
            pltpu.VMEM_SHARED((_NPAD, _D0), jnp.float32),
            pltpu.SemaphoreType.DMA,
        ],
    )(_sc_scatter_body)
    return gather, scatter


def _sc_gather_body(tab, idx2, out, idxb, rowb, sem):
    wid = lax.axis_index("s") * _NC + lax.axis_index("c")
    base_row = wid * _GRPT
    pltpu.sync_copy(idx2.at[pl.ds(base_row, _GRPT)], idxb.at[pl.ds(0, _GRPT)])

    @pl.when(wid < _GEXTRA)
    def _():
        pltpu.sync_copy(idx2.at[pl.ds(_GRPT * _NW + wid, 1)],
                        idxb.at[pl.ds(_GRPT, 1)])

    def body(g, carry):
        row = g * _GR
        copies = [
            pltpu.async_copy(tab.at[idxb.at[row + k]],
                             rowb.at[pl.ds(k * _L, _L)], sem)
            for k in range(_GR)
        ]
        for cp in copies:
            cp.wait()
        pltpu.sync_copy(rowb.at[pl.ds(0, _GR * _L), pl.ds(0, _D)],
                        out.at[pl.ds((base_row + row) * _L, _GR * _L),
                               pl.ds(0, _D)])
        return carry

    lax.fori_loop(0, _GNIT, body, 0)

    @pl.when(wid < _GEXTRA)
    def _():
        pltpu.async_copy(tab.at[idxb.at[_GRPT]],
                         rowb.at[pl.ds(0, _L)], sem).wait()
        pltpu.sync_copy(rowb.at[pl.ds(0, _L), pl.ds(0, _D)],
                        out.at[pl.ds((_GRPT * _NW + wid) * _L, _L),
                               pl.ds(0, _D)])


def _sc_scatter_body(y, dst2, z, o0, o1, idxb, yb, acc, sem):
    c = lax.axis_index("c")
    s = lax.axis_index("s")
    r0 = s * _RPT
    base_row = s * _SRPT
    nblk = jnp.where(s == _NS - 1, _SNIT_LAST, _SNIT)

    pltpu.sync_copy(z.at[pl.ds(r0, _RPT)], acc.at[pl.ds(r0, _RPT)])
    plsc.subcore_barrier()

    def _scatter_from(cbase):
        def body(g, carry):
            row = base_row + g * _SGR
            pltpu.sync_copy(dst2.at[pl.ds(row, _SGR)], idxb)
            pltpu.sync_copy(y.at[pl.ds(row * _L, _SGR * _L), pl.ds(cbase, _D0)],
                            yb)
            copies = [
                pltpu.async_copy(yb.at[pl.ds(k * _L, _L)],
                                 acc.at[idxb.at[k]], sem, add=True)
                for k in range(_SGR)
            ]
            for cp in copies:
                cp.wait()
            return carry

        lax.fori_loop(0, nblk, body, 0)

        @pl.when(s == _NS - 1)
        def _():
            row = _EROWS - 2
            pltpu.sync_copy(dst2.at[pl.ds(row, 2)], idxb.at[pl.ds(0, 2)])
            pltpu.sync_copy(y.at[pl.ds(row * _L, 2 * _L), pl.ds(cbase, _D0)],
                            yb.at[pl.ds(0, 2 * _L)])
            copies = [
                pltpu.async_copy(yb.at[pl.ds(k * _L, _L)],
                                 acc.at[idxb.at[k]], sem, add=True)
                for k in range(2)
            ]
            for cp in copies:
                cp.wait()

    @pl.when(c == 0)
    def _():
        _scatter_from(0)

    @pl.when(c == 1)
    def _():
        _scatter_from(_D0)

    plsc.subcore_barrier()

    @pl.when(c == 0)
    def _():
        pltpu.sync_copy(acc.at[pl.ds(r0, _RPT)], o0.at[pl.ds(r0, _RPT)])

    @pl.when(c == 1)
    def _():
        pltpu.sync_copy(acc.at[pl.ds(r0, _RPT)], o1.at[pl.ds(r0, _RPT)])


_BE = 6400
_INV_SQRT3 = 1.0 / math.sqrt(3.0)


def _bf(x):
    return x.astype(jnp.bfloat16)


def _selperm():
    cols = np.concatenate([np.arange(16), 16 + 3 * np.arange(8),
                           17 + 3 * np.arange(8), 18 + 3 * np.arange(8)])
    sel = np.zeros((40, _DW), np.float32)
    sel[np.arange(40), cols] = 1.0
    return jnp.asarray(sel)


def _tc_body(attr, cwb, srcb, i19, selp, w1t, w2t, w3t, wst, wvt, y01):
    at = lax.dot_general(_bf(i19[:]), _bf(attr[...]), (((1,), (1,)), ((), ())),
                         preferred_element_type=jnp.float32)
    cw = cwb[...].reshape(1, _BE)
    st = lax.dot_general(_bf(selp[:, 0:40]), _bf(srcb[...][:, 0:40]),
                         (((1,), (1,)), ((), ())),
                         preferred_element_type=jnp.float32)
    inv = at[0:16, :]
    h = jnp.maximum(
        jnp.dot(_bf(w1t[:]), _bf(inv), preferred_element_type=jnp.float32) * 0.25,
        0.0)
    h = jnp.maximum(
        jnp.dot(_bf(w2t[:]), _bf(h), preferred_element_type=jnp.float32) * 0.125,
        0.0)
    f = jnp.dot(_bf(w3t[:]), _bf(h), preferred_element_type=jnp.float32) * 0.125
    fs = f[0:24, :]
    fv = f[24:48, :]
    ex = at[16:17, :]
    ey = at[17:18, :]
    ez = at[18:19, :]
    s_ = st[0:16, :]
    vx = st[16:24, :]
    vy = st[24:32, :]
    vz = st[32:40, :]
    tp0 = (vx * ex + vy * ey + vz * ez) * _INV_SQRT3
    ms = jnp.concatenate([tp0, s_], axis=0) * fs * cw
    mx = jnp.concatenate([s_ * ex, vx], axis=0) * fv * cw
    my = jnp.concatenate([s_ * ey, vy], axis=0) * fv * cw
    mz = jnp.concatenate([s_ * ez, vz], axis=0) * fv * cw
    ys = jnp.dot(_bf(wst[:]), _bf(ms), preferred_element_type=jnp.float32)
    yx = jnp.dot(_bf(wvt[:]), _bf(mx), preferred_element_type=jnp.float32)
    yy = jnp.dot(_bf(wvt[:]), _bf(my), preferred_element_type=jnp.float32)
    yz = jnp.dot(_bf(wvt[:]), _bf(mz), preferred_element_type=jnp.float32)
    yt = jnp.concatenate([ys, yx, yy, yz], axis=0)
    y01[...] = lax.dot_general(yt, selp[:], (((0,), (0,)), ((), ())),
                               preferred_element_type=jnp.float32)


_tc_compute = pl.pallas_call(
    _tc_body,
    grid=(_E // _BE,),
    in_specs=[
        pl.BlockSpec((_BE, 19), lambda i: (i, 0)),
        pl.BlockSpec((1, _BE // _L, _L), lambda i: (i, 0, 0)),
        pl.BlockSpec((_BE, _DW), lambda i: (i, 0)),
        pl.BlockSpec((19, 19), lambda i: (0, 0)),
        pl.BlockSpec((40, _DW), lambda i: (0, 0)),
        pl.BlockSpec((64, 16), lambda i: (0, 0)),
        pl.BlockSpec((64, 64), lambda i: (0, 0)),
        pl.BlockSpec((48, 64), lambda i: (0, 0)),
        pl.BlockSpec((16, 24), lambda i: (0, 0)),
        pl.BlockSpec((8, 24), lambda i: (0, 0)),
    ],
    out_specs=pl.BlockSpec((_BE, 128), lambda i: (i, 0)),
    out_shape=jax.ShapeDtypeStruct((_E, 128), jnp.float32),
    compiler_params=pltpu.CompilerParams(fuse_transposed_lhs_in_matmul=True),
)


def kernel(edge_src, edge_dst, edge_weight_cutoff, edge_attr, node_feat,
           W1, W2, W3, Ws, Wv):
    node_tab = jnp.concatenate(
        [node_feat, jnp.zeros((_N, _DW - 40), jnp.float32)], axis=1)
    src_idx = edge_src.astype(jnp.int32).reshape(_EROWS, _L)
    dst_idx = edge_dst.astype(jnp.int32).reshape(_EROWS, _L)

    sc_gather, sc_scatter = _make_sc_kernels()
    src_feat = sc_gather(node_tab, src_idx)
    cw2 = edge_weight_cutoff.reshape(_E // _BE, _BE // _L, _L)
    y = _tc_compute(edge_attr, cw2, src_feat,
                    jnp.eye(19, dtype=jnp.float32), _selperm(),
                    W1.T, W2.T, W3.T, Ws.T, Wv.T)
    z = jnp.zeros((_NPAD, _D0), jnp.float32)
    o0, o1 = sc_scatter(y, dst_idx, z)

    return jnp.concatenate([o0[:_N, :], o1[:_N, :16]], axis=1)


# speedup vs baseline: 1.1079x; 1.0297x over previous
import functools
import math

import numpy as np

import jax
import jax.numpy as jnp
from jax import lax
from jax.experimental import pallas as pl
from jax.experimental.pallas import tpu as pltpu
from jax.experimental.pallas import tpu_sc as plsc

_N = 50000
_E = 800000
_D = 48
_DW = 128
_TW = 64
_D0 = 24
_D1 = 16
_NC = 2
_NS = 16
_NW = _NC * _NS
_NPAD = 50176
_RPT = _NPAD // _NS

_L = 128
_EROWS = _E // _L
_GRPT = _EROWS // _NW
_GEXTRA = _EROWS - _GRPT * _NW
_GR = 5
_GNIT = _GRPT // _GR
_SGR = 8
_SRPT = 392
_SNIT = 49
_SNIT_LAST = 46


@functools.cache
def _make_sc_kernels():
    mesh = plsc.VectorSubcoreMesh(
        core_axis_name="c", subcore_axis_name="s",
        num_cores=_NC, num_subcores=_NS,
    )
    params = pltpu.CompilerParams(use_tc_tiling_on_sc=False)
    gather = functools.partial(
        pl.kernel,
        out_type=jax.ShapeDtypeStruct((_E, _DW), jnp.float32),
        mesh=mesh,
        compiler_params=params,
        scratch_types=[
            pltpu.VMEM((_GRPT + 1, _L), jnp.int32),
            pltpu.VMEM((_GR * _L, _TW), jnp.float32),
            pltpu.SemaphoreType.DMA,
        ],
    )(_sc_gather_body)
    scatter = functools.partial(
        pl.kernel,
        out_type=(
            jax.ShapeDtypeStruct((_NPAD, _D0), jnp.float32),
            jax.ShapeDtypeStruct((_NPAD, _D0), jnp.float32),
        ),
        mesh=mesh,
        compiler_params=params,
        scratch_types=[
            pltpu.VMEM((_SGR, _L), jnp.int32),
            pltpu.VMEM((_SGR * _L, _D0), jnp.float32),
            pltpu.VMEM_SHARED((_NPAD, _D0), jnp.float32),
            pltpu.SemaphoreType.DMA,
        ],
    )(_sc_scatter_body)
    return gather, scatter


def _sc_gather_body(tab, idx2, out, idxb, rowb, sem):
    wid = lax.axis_index("s") * _NC + lax.axis_index("c")
    base_row = wid * _GRPT
    pltpu.sync_copy(idx2.at[pl.ds(base_row, _GRPT)], idxb.at[pl.ds(0, _GRPT)])

    @pl.when(wid < _GEXTRA)
    def _():
        pltpu.sync_copy(idx2.at[pl.ds(_GRPT * _NW + wid, 1)],
                        idxb.at[pl.ds(_GRPT, 1)])

    def body(g, carry):
        row = g * _GR
        copies = [
            pltpu.async_copy(tab.at[idxb.at[row + k]],
                             rowb.at[pl.ds(k * _L, _L)], sem)
            for k in range(_GR)
        ]
        for cp in copies:
            cp.wait()
        pltpu.sync_copy(rowb.at[pl.ds(0, _GR * _L), pl.ds(0, _D)],
                        out.at[pl.ds((base_row + row) * _L, _GR * _L),
                               pl.ds(0, _D)])
        return carry

    lax.fori_loop(0, _GNIT, body, 0)

    @pl.when(wid < _GEXTRA)
    def _():
        pltpu.async_copy(tab.at[idxb.at[_GRPT]],
                         rowb.at[pl.ds(0, _L)], sem).wait()
        pltpu.sync_copy(rowb.at[pl.ds(0, _L), pl.ds(0, _D)],
                        out.at[pl.ds((_GRPT * _NW + wid) * _L, _L),
                               pl.ds(0, _D)])


def _sc_scatter_body(y, dst2, z, o0, o1, idxb, yb, acc, sem):
    c = lax.axis_index("c")
    s = lax.axis_index("s")
    r0 = s * _RPT
    base_row = s * _SRPT
    nblk = jnp.where(s == _NS - 1, _SNIT_LAST, _SNIT)

    pltpu.sync_copy(z.at[pl.ds(r0, _RPT)], acc.at[pl.ds(r0, _RPT)])
    plsc.subcore_barrier()

    def _scatter_from(cbase):
        def body(g, carry):
            row = base_row + g * _SGR
            pltpu.sync_copy(dst2.at[pl.ds(row, _SGR)], idxb)
            pltpu.sync_copy(y.at[pl.ds(row * _L, _SGR * _L), pl.ds(cbase, _D0)],
                            yb)
            copies = [
                pltpu.async_copy(yb.at[pl.ds(k * _L, _L)],
                                 acc.at[idxb.at[k]], sem, add=True)
                for k in range(_SGR)
            ]
            for cp in copies:
                cp.wait()
            return carry

        lax.fori_loop(0, nblk, body, 0)

        @pl.when(s == _NS - 1)
        def _():
            row = _EROWS - 2
            pltpu.sync_copy(dst2.at[pl.ds(row, 2)], idxb.at[pl.ds(0, 2)])
            pltpu.sync_copy(y.at[pl.ds(row * _L, 2 * _L), pl.ds(cbase, _D0)],
                            yb.at[pl.ds(0, 2 * _L)])
            copies = [
                pltpu.async_copy(yb.at[pl.ds(k * _L, _L)],
                                 acc.at[idxb.at[k]], sem, add=True)
                for k in range(2)
            ]
            for cp in copies:
                cp.wait()

    @pl.when(c == 0)
    def _():
        _scatter_from(0)

    @pl.when(c == 1)
    def _():
        _scatter_from(_D0)

    plsc.subcore_barrier()

    @pl.when(c == 0)
    def _():
        pltpu.sync_copy(acc.at[pl.ds(r0, _RPT)], o0.at[pl.ds(r0, _RPT)])

    @pl.when(c == 1)
    def _():
        pltpu.sync_copy(acc.at[pl.ds(r0, _RPT)], o1.at[pl.ds(r0, _RPT)])


_BE = 6400
_INV_SQRT3 = 1.0 / math.sqrt(3.0)


def _bf(x):
    return x.astype(jnp.bfloat16)


def _selperm():
    cols = np.concatenate([np.arange(16), 16 + 3 * np.arange(8),
                           17 + 3 * np.arange(8), 18 + 3 * np.arange(8)])
    sel = np.zeros((40, _DW), np.float32)
    sel[np.arange(40), cols] = 1.0
    return jnp.asarray(sel)


def _tc_body(attr, cwb, srcb, i19, selp, w1t, w2t, w3t, wst, wvt, y01):
    at = lax.dot_general(_bf(i19[:]), _bf(attr[...]), (((1,), (1,)), ((), ())),
                         preferred_element_type=jnp.float32)
    cw = cwb[...].reshape(1, _BE)
    st = lax.dot_general(_bf(selp[:, 0:40]), _bf(srcb[...][:, 0:40]),
                         (((1,), (1,)), ((), ())),
                         preferred_element_type=jnp.float32)
    inv = at[0:16, :]
    h = jnp.maximum(
        jnp.dot(_bf(w1t[:]), _bf(inv), preferred_element_type=jnp.float32) * 0.25,
        0.0)
    h = jnp.maximum(
        jnp.dot(_bf(w2t[:]), _bf(h), preferred_element_type=jnp.float32) * 0.125,
        0.0)
    f = jnp.dot(_bf(w3t[:]), _bf(h), preferred_element_type=jnp.float32) * 0.125
    fs = f[0:24, :]
    fv = f[24:48, :]
    ex = at[16:17, :]
    ey = at[17:18, :]
    ez = at[18:19, :]
    s_ = st[0:16, :]
    vx = st[16:24, :]
    vy = st[24:32, :]
    vz = st[32:40, :]
    tp0 = (vx * ex + vy * ey + vz * ez) * _INV_SQRT3
    ms = jnp.concatenate([tp0, s_], axis=0) * fs * cw
    mx = jnp.concatenate([s_ * ex, vx], axis=0) * fv * cw
    my = jnp.concatenate([s_ * ey, vy], axis=0) * fv * cw
    mz = jnp.concatenate([s_ * ez, vz], axis=0) * fv * cw
    ys = jnp.dot(_bf(wst[:]), _bf(ms), preferred_element_type=jnp.float32)
    yx = jnp.dot(_bf(wvt[:]), _bf(mx), preferred_element_type=jnp.float32)
    yy = jnp.dot(_bf(wvt[:]), _bf(my), preferred_element_type=jnp.float32)
    yz = jnp.dot(_bf(wvt[:]), _bf(mz), preferred_element_type=jnp.float32)
    yt = jnp.concatenate([ys, yx, yy, yz], axis=0)
    y01[...] = lax.dot_general(yt, selp[:], (((0,), (0,)), ((), ())),
                               preferred_element_type=jnp.float32)


_tc_compute = pl.pallas_call(
    _tc_body,
    grid=(_E // _BE,),
    in_specs=[
        pl.BlockSpec((_BE, 19), lambda i: (i, 0)),
        pl.BlockSpec((1, _BE // _L, _L), lambda i: (i, 0, 0)),
        pl.BlockSpec((_BE, _DW), lambda i: (i, 0)),
        pl.BlockSpec((19, 19), lambda i: (0, 0)),
        pl.BlockSpec((40, _DW), lambda i: (0, 0)),
        pl.BlockSpec((64, 16), lambda i: (0, 0)),
        pl.BlockSpec((64, 64), lambda i: (0, 0)),
        pl.BlockSpec((48, 64), lambda i: (0, 0)),
        pl.BlockSpec((16, 24), lambda i: (0, 0)),
        pl.BlockSpec((8, 24), lambda i: (0, 0)),
    ],
    out_specs=pl.BlockSpec((_BE, 128), lambda i: (i, 0)),
    out_shape=jax.ShapeDtypeStruct((_E, 128), jnp.float32),
    compiler_params=pltpu.CompilerParams(fuse_transposed_lhs_in_matmul=True),
)


def kernel(edge_src, edge_dst, edge_weight_cutoff, edge_attr, node_feat,
           W1, W2, W3, Ws, Wv):
    node_tab = jnp.concatenate(
        [node_feat, jnp.zeros((_N, _TW - 40), jnp.float32)], axis=1)
    src_idx = edge_src.astype(jnp.int32).reshape(_EROWS, _L)
    dst_idx = edge_dst.astype(jnp.int32).reshape(_EROWS, _L)

    sc_gather, sc_scatter = _make_sc_kernels()
    src_feat = sc_gather(node_tab, src_idx)
    cw2 = edge_weight_cutoff.reshape(_E // _BE, _BE // _L, _L)
    y = _tc_compute(edge_attr, cw2, src_feat,
                    jnp.eye(19, dtype=jnp.float32), _selperm(),
                    W1.T, W2.T, W3.T, Ws.T, Wv.T)
    z = jnp.zeros((_NPAD, _D0), jnp.float32)
    o0, o1 = sc_scatter(y, dst_idx, z)

    return jnp.concatenate([o0[:_N, :], o1[:_N, :16]], axis=1)


# speedup vs baseline: 1.1305x; 1.0204x over previous
import functools
import math

import numpy as np

import jax
import jax.numpy as jnp
from jax import lax
from jax.experimental import pallas as pl
from jax.experimental.pallas import tpu as pltpu
from jax.experimental.pallas import tpu_sc as plsc

_N = 50000
_E = 800000
_D = 48
_DW = 128
_TW = 64
_D0 = 24
_D1 = 16
_NC = 2
_NS = 16
_NW = _NC * _NS
_NPAD = 50176
_RPT = _NPAD // _NS

_L = 128
_EROWS = _E // _L
_GRPT = _EROWS // _NW
_GEXTRA = _EROWS - _GRPT * _NW
_GR = 5
_GNIT = _GRPT // _GR
_SGR = 16
_SRPT = 400
_SNIT = 25
_SNIT_LAST = 15
_STAIL = 10


@functools.cache
def _make_sc_kernels():
    mesh = plsc.VectorSubcoreMesh(
        core_axis_name="c", subcore_axis_name="s",
        num_cores=_NC, num_subcores=_NS,
    )
    params = pltpu.CompilerParams(use_tc_tiling_on_sc=False)
    gather = functools.partial(
        pl.kernel,
        out_type=jax.ShapeDtypeStruct((_E, _DW), jnp.float32),
        mesh=mesh,
        compiler_params=params,
        scratch_types=[
            pltpu.VMEM((_GRPT + 1, _L), jnp.int32),
            pltpu.VMEM((_GR * _L, _TW), jnp.float32),
            pltpu.SemaphoreType.DMA,
        ],
    )(_sc_gather_body)
    scatter = functools.partial(
        pl.kernel,
        out_type=(
            jax.ShapeDtypeStruct((_NPAD, _D0), jnp.float32),
            jax.ShapeDtypeStruct((_NPAD, _D0), jnp.float32),
        ),
        mesh=mesh,
        compiler_params=params,
        scratch_types=[
            pltpu.VMEM((_SGR, _L), jnp.int32),
            pltpu.VMEM((_SGR * _L, _D0), jnp.float32),
            pltpu.VMEM_SHARED((_NPAD, _D0), jnp.float32),
            pltpu.SemaphoreType.DMA,
        ],
    )(_sc_scatter_body)
    return gather, scatter


def _sc_gather_body(tab, idx2, out, idxb, rowb, sem):
    wid = lax.axis_index("s") * _NC + lax.axis_index("c")
    base_row = wid * _GRPT
    pltpu.sync_copy(idx2.at[pl.ds(base_row, _GRPT)], idxb.at[pl.ds(0, _GRPT)])

    @pl.when(wid < _GEXTRA)
    def _():
        pltpu.sync_copy(idx2.at[pl.ds(_GRPT * _NW + wid, 1)],
                        idxb.at[pl.ds(_GRPT, 1)])

    def body(g, carry):
        row = g * _GR
        copies = [
            pltpu.async_copy(tab.at[idxb.at[row + k]],
                             rowb.at[pl.ds(k * _L, _L)], sem)
            for k in range(_GR)
        ]
        for cp in copies:
            cp.wait()
        pltpu.sync_copy(rowb.at[pl.ds(0, _GR * _L), pl.ds(0, _D)],
                        out.at[pl.ds((base_row + row) * _L, _GR * _L),
                               pl.ds(0, _D)])
        return carry

    lax.fori_loop(0, _GNIT, body, 0)

    @pl.when(wid < _GEXTRA)
    def _():
        pltpu.async_copy(tab.at[idxb.at[_GRPT]],
                         rowb.at[pl.ds(0, _L)], sem).wait()
        pltpu.sync_copy(rowb.at[pl.ds(0, _L), pl.ds(0, _D)],
                        out.at[pl.ds((_GRPT * _NW + wid) * _L, _L),
                               pl.ds(0, _D)])


def _sc_scatter_body(y, dst2, z, o0, o1, idxb, yb, acc, sem):
    c = lax.axis_index("c")
    s = lax.axis_index("s")
    r0 = s * _RPT
    base_row = s * _SRPT
    nblk = jnp.where(s == _NS - 1, _SNIT_LAST, _SNIT)

    pltpu.sync_copy(z.at[pl.ds(r0, _RPT)], acc.at[pl.ds(r0, _RPT)])
    plsc.subcore_barrier()

    def _scatter_from(cbase):
        def body(g, carry):
            row = base_row + g * _SGR
            pltpu.sync_copy(dst2.at[pl.ds(row, _SGR)], idxb)
            pltpu.sync_copy(y.at[pl.ds(row * _L, _SGR * _L), pl.ds(cbase, _D0)],
                            yb)
            copies = [
                pltpu.async_copy(yb.at[pl.ds(k * _L, _L)],
                                 acc.at[idxb.at[k]], sem, add=True)
                for k in range(_SGR)
            ]
            for cp in copies:
                cp.wait()
            return carry

        lax.fori_loop(0, nblk, body, 0)

        @pl.when(s == _NS - 1)
        def _():
            row = _EROWS - _STAIL
            pltpu.sync_copy(dst2.at[pl.ds(row, _STAIL)],
                            idxb.at[pl.ds(0, _STAIL)])
            pltpu.sync_copy(
                y.at[pl.ds(row * _L, _STAIL * _L), pl.ds(cbase, _D0)],
                yb.at[pl.ds(0, _STAIL * _L)])
            copies = [
                pltpu.async_copy(yb.at[pl.ds(k * _L, _L)],
                                 acc.at[idxb.at[k]], sem, add=True)
                for k in range(_STAIL)
            ]
            for cp in copies:
                cp.wait()

    @pl.when(c == 0)
    def _():
        _scatter_from(0)

    @pl.when(c == 1)
    def _():
        _scatter_from(_D0)

    plsc.subcore_barrier()

    @pl.when(c == 0)
    def _():
        pltpu.sync_copy(acc.at[pl.ds(r0, _RPT)], o0.at[pl.ds(r0, _RPT)])

    @pl.when(c == 1)
    def _():
        pltpu.sync_copy(acc.at[pl.ds(r0, _RPT)], o1.at[pl.ds(r0, _RPT)])


_BE = 6400
_INV_SQRT3 = 1.0 / math.sqrt(3.0)


def _bf(x):
    return x.astype(jnp.bfloat16)


def _selperm():
    cols = np.concatenate([np.arange(16), 16 + 3 * np.arange(8),
                           17 + 3 * np.arange(8), 18 + 3 * np.arange(8)])
    sel = np.zeros((40, _DW), np.float32)
    sel[np.arange(40), cols] = 1.0
    return jnp.asarray(sel)


def _tc_body(attr, cwb, srcb, i19, selp, w1t, w2t, w3t, wst, wvt, y01):
    at = lax.dot_general(_bf(i19[:]), _bf(attr[...]), (((1,), (1,)), ((), ())),
                         preferred_element_type=jnp.float32)
    cw = cwb[...].reshape(1, _BE)
    st = lax.dot_general(_bf(selp[:, 0:40]), _bf(srcb[...][:, 0:40]),
                         (((1,), (1,)), ((), ())),
                         preferred_element_type=jnp.float32)
    inv = at[0:16, :]
    h = jnp.maximum(
        jnp.dot(_bf(w1t[:]), _bf(inv), preferred_element_type=jnp.float32) * 0.25,
        0.0)
    h = jnp.maximum(
        jnp.dot(_bf(w2t[:]), _bf(h), preferred_element_type=jnp.float32) * 0.125,
        0.0)
    f = jnp.dot(_bf(w3t[:]), _bf(h), preferred_element_type=jnp.float32) * 0.125
    fs = f[0:24, :]
    fv = f[24:48, :]
    ex = at[16:17, :]
    ey = at[17:18, :]
    ez = at[18:19, :]
    s_ = st[0:16, :]
    vx = st[16:24, :]
    vy = st[24:32, :]
    vz = st[32:40, :]
    tp0 = (vx * ex + vy * ey + vz * ez) * _INV_SQRT3
    ms = jnp.concatenate([tp0, s_], axis=0) * fs * cw
    mx = jnp.concatenate([s_ * ex, vx], axis=0) * fv * cw
    my = jnp.concatenate([s_ * ey, vy], axis=0) * fv * cw
    mz = jnp.concatenate([s_ * ez, vz], axis=0) * fv * cw
    ys = jnp.dot(_bf(wst[:]), _bf(ms), preferred_element_type=jnp.float32)
    yx = jnp.dot(_bf(wvt[:]), _bf(mx), preferred_element_type=jnp.float32)
    yy = jnp.dot(_bf(wvt[:]), _bf(my), preferred_element_type=jnp.float32)
    yz = jnp.dot(_bf(wvt[:]), _bf(mz), preferred_element_type=jnp.float32)
    yt = jnp.concatenate([ys, yx, yy, yz], axis=0)
    y01[...] = lax.dot_general(yt, selp[:], (((0,), (0,)), ((), ())),
                               preferred_element_type=jnp.float32)


_tc_compute = pl.pallas_call(
    _tc_body,
    grid=(_E // _BE,),
    in_specs=[
        pl.BlockSpec((_BE, 19), lambda i: (i, 0)),
        pl.BlockSpec((1, _BE // _L, _L), lambda i: (i, 0, 0)),
        pl.BlockSpec((_BE, _DW), lambda i: (i, 0)),
        pl.BlockSpec((19, 19), lambda i: (0, 0)),
        pl.BlockSpec((40, _DW), lambda i: (0, 0)),
        pl.BlockSpec((64, 16), lambda i: (0, 0)),
        pl.BlockSpec((64, 64), lambda i: (0, 0)),
        pl.BlockSpec((48, 64), lambda i: (0, 0)),
        pl.BlockSpec((16, 24), lambda i: (0, 0)),
        pl.BlockSpec((8, 24), lambda i: (0, 0)),
    ],
    out_specs=pl.BlockSpec((_BE, 128), lambda i: (i, 0)),
    out_shape=jax.ShapeDtypeStruct((_E, 128), jnp.float32),
    compiler_params=pltpu.CompilerParams(fuse_transposed_lhs_in_matmul=True),
)


def kernel(edge_src, edge_dst, edge_weight_cutoff, edge_attr, node_feat,
           W1, W2, W3, Ws, Wv):
    node_tab = jnp.concatenate(
        [node_feat, jnp.zeros((_N, _TW - 40), jnp.float32)], axis=1)
    src_idx = edge_src.astype(jnp.int32).reshape(_EROWS, _L)
    dst_idx = edge_dst.astype(jnp.int32).reshape(_EROWS, _L)

    sc_gather, sc_scatter = _make_sc_kernels()
    src_feat = sc_gather(node_tab, src_idx)
    cw2 = edge_weight_cutoff.reshape(_E // _BE, _BE // _L, _L)
    y = _tc_compute(edge_attr, cw2, src_feat,
                    jnp.eye(19, dtype=jnp.float32), _selperm(),
                    W1.T, W2.T, W3.T, Ws.T, Wv.T)
    z = jnp.zeros((_NPAD, _D0), jnp.float32)
    o0, o1 = sc_scatter(y, dst_idx, z)

    return jnp.concatenate([o0[:_N, :], o1[:_N, :16]], axis=1)
